# baseline (speedup 1.0000x reference)
import jax
import jax.numpy as jnp
from jax import lax
from jax.experimental import pallas as pl
from jax.experimental.pallas import tpu as pltpu
from jax.experimental.pallas import tpu_sc as plsc

N = 10000
E = 320000
D = 128
H = 128

NC = 2
NS = 16
NW = NC * NS
C = 80
NCHUNK = 126
NCH0 = 240
NCH1 = 12
EP = NW * NCHUNK * C
NP = 10112
ZR = NP // NS
NBUF = 3
NIDX = 6
RB = 1000


NH = 6
ZR8 = 640


def _sc_hist_body(d1_hbm, out_hbm, *scr):
    dbufs = scr[0:NH]
    ones_v = scr[NH]
    stage = scr[NH + 1]
    acc = scr[NH + 2]
    sem_i = scr[NH + 3]

    cid = lax.axis_index("c")
    sid = lax.axis_index("s")
    wid = cid * NS + sid
    base = wid * (NCHUNK * C)
    for k in range(C // 16):
        ones_v[pl.ds(16 * k, 16)] = jnp.full((16,), 1.0, jnp.float32)
    for k in range(ZR8 // 16):
        stage[pl.ds(16 * k, 16)] = jnp.zeros((16,), jnp.float32)
    off0 = pl.multiple_of(sid * ZR, 8)
    pltpu.sync_copy(stage.at[pl.ds(0, ZR)], acc.at[pl.ds(off0, ZR)])

    def fire(j, m):
        src = d1_hbm.at[pl.ds(pl.multiple_of(base + j * C, 16), C)]
        pltpu.async_copy(src, dbufs[m], sem_i.at[m])

    def wait(j, m):
        src = d1_hbm.at[pl.ds(pl.multiple_of(base + j * C, 16), C)]
        pltpu.make_async_copy(src, dbufs[m], sem_i.at[m]).wait()

    for t in range(NH):
        fire(t, t)
    plsc.subcore_barrier()

    def step(j, t, last):
        wait(j, t)
        pltpu.sync_copy(ones_v, acc.at[dbufs[t]], add=True)
        if (not last) or (j + NH < NCHUNK):
            fire(j + NH, t)

    def body(g, carry):
        for t in range(NH):
            step(g * NH + t, t, False)
        return carry

    lax.fori_loop(0, NCHUNK // NH - 1, body, 0)
    for j in range(NCHUNK - NH, NCHUNK):
        step(j, j % NH, True)

    plsc.subcore_barrier()
    pltpu.sync_copy(acc.at[pl.ds(off0, ZR)], stage.at[pl.ds(0, ZR)])
    pltpu.sync_copy(stage.at[pl.ds(0, ZR)],
                    out_hbm.at[pl.ds(cid * NP + off0, ZR)])


def _sc_hist(d1):
    mesh = plsc.VectorSubcoreMesh(core_axis_name="c", subcore_axis_name="s")
    scratches = (
        [pltpu.VMEM((C,), jnp.int32) for _ in range(NH)]
        + [
            pltpu.VMEM((C,), jnp.float32),
            pltpu.VMEM((ZR8,), jnp.float32),
            pltpu.VMEM_SHARED((NP,), jnp.float32),
            pltpu.SemaphoreType.DMA((NH,)),
        ]
    )
    return pl.kernel(
        _sc_hist_body,
        out_type=jax.ShapeDtypeStruct((NC * NP,), jnp.float32),
        mesh=mesh,
        scratch_types=scratches,
    )(d1)


def _sc_agg_body(s1_hbm, d1_hbm, hl_hbm, out_hbm, *scr):
    sbufs = scr[0:NIDX]
    dbufs = scr[NIDX:2 * NIDX]
    rbufs = scr[2 * NIDX:2 * NIDX + NBUF]
    acc = scr[2 * NIDX + NBUF]
    sem_i = scr[2 * NIDX + NBUF + 1]
    sem_g = scr[2 * NIDX + NBUF + 2]
    sem_s = scr[2 * NIDX + NBUF + 3]

    cid = lax.axis_index("c")
    sid = lax.axis_index("s")
    nch = jnp.where(cid == 0, NCH0, NCH1)
    base = jnp.where(cid == 0, sid * (NCH0 * C),
                     NS * (NCH0 * C) + sid * (NCH1 * C))
    row0 = pl.multiple_of(sid * ZR, 8)

    def zfill(i, carry):
        for k in range(H // 16):
            rbufs[0][i, pl.ds(16 * k, 16)] = jnp.zeros((16,), jnp.float32)
        return carry

    lax.fori_loop(0, C, zfill, 0)
    for q in range(ZR // C):
        pltpu.sync_copy(rbufs[0], acc.at[pl.ds(row0 + C * q, C)])
    zr_tail = ZR - (ZR // C) * C
    if zr_tail:
        pltpu.sync_copy(rbufs[0].at[pl.ds(0, zr_tail)],
                        acc.at[pl.ds(row0 + (ZR // C) * C, zr_tail)])

    def _chunk(ref, j):
        return ref.at[pl.ds(pl.multiple_of(base + j * C, 16), C)]

    def fire_idx(j, m):
        pltpu.async_copy(_chunk(s1_hbm, j), sbufs[m], sem_i.at[m])
        pltpu.async_copy(_chunk(d1_hbm, j), dbufs[m], sem_i.at[m])

    def wait_idx(j, m):
        pltpu.make_async_copy(_chunk(s1_hbm, j), sbufs[m],
                              sem_i.at[m]).wait()
        pltpu.make_async_copy(_chunk(d1_hbm, j), dbufs[m],
                              sem_i.at[m]).wait()

    def fire_gather(j, m, r):
        pltpu.async_copy(hl_hbm.at[sbufs[m]], rbufs[r], sem_g.at[r])

    def wait_gather(j, m, r):
        pltpu.make_async_copy(hl_hbm.at[sbufs[m]], rbufs[r],
                              sem_g.at[r]).wait()

    def fire_scatter(j, m, r):
        pltpu.async_copy(rbufs[r], acc.at[dbufs[m]],
                         sem_s.at[r], add=True)

    def wait_scatter(j, m, r):
        pltpu.make_async_copy(rbufs[r], acc.at[dbufs[m]],
                              sem_s.at[r]).wait()

    def step(j, t, do_wait_sc, do_next_gather, do_fire_idx):
        m, r = t, t % NBUF
        wait_gather(j, m, r)
        fire_scatter(j, m, r)
        if do_next_gather:
            km, kr = (t + 2) % NIDX, (t + 2) % NBUF
            if do_wait_sc:
                wait_scatter(j - 1, (t - 1) % NIDX, kr)
            wait_idx(j + 2, km)
            fire_gather(j + 2, km, kr)
        if do_fire_idx:
            fire_idx(j + NIDX - 2, (t - 2) % NIDX)

    for t in range(NIDX):
        fire_idx(t, t)
    plsc.subcore_barrier()
    for t in range(2):
        wait_idx(t, t)
        fire_gather(t, t, t)

    G = NIDX
    for t in range(G):
        step(t, t, t >= 1, True, t >= 2)

    def body(g, carry):
        for t in range(G):
            step(g * G + t, t, True, True, True)
        return carry

    lax.fori_loop(1, nch // G - 1, body, 0)
    for t in range(G):
        step(nch - G + t, t, t < G - 2, t < G - 2, t < 2)
    for t in range(G - NBUF, G):
        wait_scatter(nch - G + t, t, t % NBUF)

    plsc.subcore_barrier()
    pltpu.sync_copy(acc.at[pl.ds(row0, ZR)],
                    out_hbm.at[cid, pl.ds(row0, ZR)])


def _sc_agg(s1, d1, hl):
    mesh = plsc.VectorSubcoreMesh(core_axis_name="c", subcore_axis_name="s")
    scratches = (
        [pltpu.VMEM((C,), jnp.int32) for _ in range(2 * NIDX)]
        + [pltpu.VMEM((C, H), jnp.float32) for _ in range(NBUF)]
        + [
            pltpu.VMEM_SHARED((NP, H), jnp.float32),
            pltpu.SemaphoreType.DMA((NIDX,)),
            pltpu.SemaphoreType.DMA((NBUF,)),
            pltpu.SemaphoreType.DMA((NBUF,)),
        ]
    )
    return pl.kernel(
        _sc_agg_body,
        out_type=jax.ShapeDtypeStruct((NC, NP, H), jnp.float32),
        mesh=mesh,
        scratch_types=scratches,
    )(s1, d1, hl)


def _tc_l1_body(degp_ref, x_ref, w_ref, hlp_ref, dis_ref):
    deg = degp_ref[0] + degp_ref[1] + 1.0
    dis = lax.rsqrt(deg)
    hl = jnp.dot(x_ref[...], w_ref[...], preferred_element_type=jnp.float32)
    hlp_ref[...] = hl * dis
    dis_ref[...] = dis


def _tc_l1(degp, x, W1):
    return pl.pallas_call(
        _tc_l1_body,
        grid=(N // RB,),
        in_specs=[
            pl.BlockSpec((NC, RB, 1), lambda i: (0, i, 0)),
            pl.BlockSpec((RB, D), lambda i: (i, 0)),
            pl.BlockSpec((D, H), lambda i: (0, 0)),
        ],
        out_specs=[
            pl.BlockSpec((RB, H), lambda i: (i, 0)),
            pl.BlockSpec((RB, 1), lambda i: (i, 0)),
        ],
        out_shape=[
            jax.ShapeDtypeStruct((N, H), jnp.float32),
            jax.ShapeDtypeStruct((N, 1), jnp.float32),
        ],
    )(degp, x, W1)


def _bn_relu(t, g, be):
    mu = jnp.mean(t, axis=0, keepdims=True)
    xc = t - mu
    var = jnp.mean(xc * xc, axis=0, keepdims=True)
    return jnp.maximum(xc * lax.rsqrt(var + 1e-5) * g + be, 0.0)


def _tc_mid_body(p_ref, hlp_ref, dis_ref, b_ref, g_ref, be_ref, w_ref,
                 out_ref):
    agg = p_ref[0, :N] + p_ref[1, :N] + hlp_ref[...]
    t = agg * dis_ref[...] + b_ref[...]
    h2 = _bn_relu(t, g_ref[...], be_ref[...])
    out_ref[...] = jnp.dot(h2, w_ref[...],
                           preferred_element_type=jnp.float32) * dis_ref[...]


def _tc_mid(p, hlp, dis, b, g, be, W):
    return pl.pallas_call(
        _tc_mid_body,
        out_shape=jax.ShapeDtypeStruct((N, H), jnp.float32),
    )(p, hlp, dis, b, g, be, W)


def _tc_final_body(p_ref, hlp_ref, dis_ref, b_ref, g_ref, be_ref, w_ref,
                   bf_ref, out_ref):
    agg = p_ref[0, :N] + p_ref[1, :N] + hlp_ref[...]
    t = agg * dis_ref[...] + b_ref[...]
    h3 = _bn_relu(t, g_ref[...], be_ref[...])
    out_ref[...] = jnp.dot(h3, w_ref[...],
                           preferred_element_type=jnp.float32) + bf_ref[...]


def _tc_final(p, hlp, dis, b, g, be, Wf, bf):
    return pl.pallas_call(
        _tc_final_body,
        out_shape=jax.ShapeDtypeStruct((N, H), jnp.float32),
    )(p, hlp, dis, b, g, be, Wf, bf)


def kernel(x, edge_index, W1, b1, g1, be1, W2, b2, g2, be2, Wf, bf):
    pad = EP - E
    s1 = jnp.concatenate(
        [edge_index[0], jnp.zeros((pad,), jnp.int32)])
    d1 = jnp.concatenate(
        [edge_index[1], jnp.full((pad,), N, jnp.int32)])
    b1r, g1r, be1r = b1.reshape(1, H), g1.reshape(1, H), be1.reshape(1, H)
    b2r, g2r, be2r = b2.reshape(1, H), g2.reshape(1, H), be2.reshape(1, H)
    bfr = bf.reshape(1, H)

    degp = _sc_hist(d1).reshape(NC, NP, 1)
    hlp1, dis = _tc_l1(degp, x, W1)
    p1 = _sc_agg(s1, d1, hlp1)
    hlp2 = _tc_mid(p1, hlp1, dis, b1r, g1r, be1r, W2)
    p2 = _sc_agg(s1, d1, hlp2)
    return _tc_final(p2, hlp2, dis, b2r, g2r, be2r, Wf, bfr)


# speedup vs baseline: 1.0182x; 1.0182x over previous
import jax
import jax.numpy as jnp
from jax import lax
from jax.experimental import pallas as pl
from jax.experimental.pallas import tpu as pltpu
from jax.experimental.pallas import tpu_sc as plsc

N = 10000
E = 320000
D = 128
H = 128

NC = 2
NS = 16
NW = NC * NS
C = 80
NCHUNK = 126
NCH0 = 228
NCH1 = 24
EP = NW * NCHUNK * C
NP = 10112
ZR = NP // NS
NBUF = 4
NIDX = 6
RB = 1000


NH = 6
ZR8 = 640


def _sc_hist_body(d1_hbm, out_hbm, *scr):
    dbufs = scr[0:NH]
    ones_v = scr[NH]
    stage = scr[NH + 1]
    acc = scr[NH + 2]
    sem_i = scr[NH + 3]

    cid = lax.axis_index("c")
    sid = lax.axis_index("s")
    wid = cid * NS + sid
    base = wid * (NCHUNK * C)
    for k in range(C // 16):
        ones_v[pl.ds(16 * k, 16)] = jnp.full((16,), 1.0, jnp.float32)
    for k in range(ZR8 // 16):
        stage[pl.ds(16 * k, 16)] = jnp.zeros((16,), jnp.float32)
    off0 = pl.multiple_of(sid * ZR, 8)
    pltpu.sync_copy(stage.at[pl.ds(0, ZR)], acc.at[pl.ds(off0, ZR)])

    def fire(j, m):
        src = d1_hbm.at[pl.ds(pl.multiple_of(base + j * C, 16), C)]
        pltpu.async_copy(src, dbufs[m], sem_i.at[m])

    def wait(j, m):
        src = d1_hbm.at[pl.ds(pl.multiple_of(base + j * C, 16), C)]
        pltpu.make_async_copy(src, dbufs[m], sem_i.at[m]).wait()

    for t in range(NH):
        fire(t, t)
    plsc.subcore_barrier()

    def step(j, t, last):
        wait(j, t)
        pltpu.sync_copy(ones_v, acc.at[dbufs[t]], add=True)
        if (not last) or (j + NH < NCHUNK):
            fire(j + NH, t)

    def body(g, carry):
        for t in range(NH):
            step(g * NH + t, t, False)
        return carry

    lax.fori_loop(0, NCHUNK // NH - 1, body, 0)
    for j in range(NCHUNK - NH, NCHUNK):
        step(j, j % NH, True)

    plsc.subcore_barrier()
    pltpu.sync_copy(acc.at[pl.ds(off0, ZR)], stage.at[pl.ds(0, ZR)])
    pltpu.sync_copy(stage.at[pl.ds(0, ZR)],
                    out_hbm.at[pl.ds(cid * NP + off0, ZR)])


def _sc_hist(d1):
    mesh = plsc.VectorSubcoreMesh(core_axis_name="c", subcore_axis_name="s")
    scratches = (
        [pltpu.VMEM((C,), jnp.int32) for _ in range(NH)]
        + [
            pltpu.VMEM((C,), jnp.float32),
            pltpu.VMEM((ZR8,), jnp.float32),
            pltpu.VMEM_SHARED((NP,), jnp.float32),
            pltpu.SemaphoreType.DMA((NH,)),
        ]
    )
    return pl.kernel(
        _sc_hist_body,
        out_type=jax.ShapeDtypeStruct((NC * NP,), jnp.float32),
        mesh=mesh,
        scratch_types=scratches,
    )(d1)


def _sc_agg_body(s1_hbm, d1_hbm, hl_hbm, out_hbm, *scr):
    sbufs = scr[0:NIDX]
    dbufs = scr[NIDX:2 * NIDX]
    rbufs = scr[2 * NIDX:2 * NIDX + NBUF]
    acc = scr[2 * NIDX + NBUF]
    sem_i = scr[2 * NIDX + NBUF + 1]
    sem_g = scr[2 * NIDX + NBUF + 2]
    sem_s = scr[2 * NIDX + NBUF + 3]

    cid = lax.axis_index("c")
    sid = lax.axis_index("s")
    nch = jnp.where(cid == 0, NCH0, NCH1)
    base = jnp.where(cid == 0, sid * (NCH0 * C),
                     NS * (NCH0 * C) + sid * (NCH1 * C))
    row0 = pl.multiple_of(sid * ZR, 8)

    def zfill(i, carry):
        for k in range(H // 16):
            rbufs[0][i, pl.ds(16 * k, 16)] = jnp.zeros((16,), jnp.float32)
        return carry

    lax.fori_loop(0, C, zfill, 0)
    for q in range(ZR // C):
        pltpu.sync_copy(rbufs[0], acc.at[pl.ds(row0 + C * q, C)])
    zr_tail = ZR - (ZR // C) * C
    if zr_tail:
        pltpu.sync_copy(rbufs[0].at[pl.ds(0, zr_tail)],
                        acc.at[pl.ds(row0 + (ZR // C) * C, zr_tail)])

    def _chunk(ref, j):
        return ref.at[pl.ds(pl.multiple_of(base + j * C, 16), C)]

    def fire_idx(j, m):
        pltpu.async_copy(_chunk(s1_hbm, j), sbufs[m], sem_i.at[m])
        pltpu.async_copy(_chunk(d1_hbm, j), dbufs[m], sem_i.at[m])

    def wait_idx(j, m):
        pltpu.make_async_copy(_chunk(s1_hbm, j), sbufs[m],
                              sem_i.at[m]).wait()
        pltpu.make_async_copy(_chunk(d1_hbm, j), dbufs[m],
                              sem_i.at[m]).wait()

    def fire_gather(j, m, r):
        pltpu.async_copy(hl_hbm.at[sbufs[m]], rbufs[r], sem_g.at[r])

    def wait_gather(j, m, r):
        pltpu.make_async_copy(hl_hbm.at[sbufs[m]], rbufs[r],
                              sem_g.at[r]).wait()

    def fire_scatter(j, m, r):
        pltpu.async_copy(rbufs[r], acc.at[dbufs[m]],
                         sem_s.at[r], add=True)

    def wait_scatter(j, m, r):
        pltpu.make_async_copy(rbufs[r], acc.at[dbufs[m]],
                              sem_s.at[r]).wait()

    def step(j, t, do_wait_sc, do_next_gather, do_fire_idx):
        m, r = t % NIDX, t % NBUF
        wait_gather(j, m, r)
        fire_scatter(j, m, r)
        if do_next_gather:
            km, kr = (t + 2) % NIDX, (t + 2) % NBUF
            if do_wait_sc:
                wait_scatter(j - 2, (t - 2) % NIDX, (t - 2) % NBUF)
            wait_idx(j + 2, km)
            fire_gather(j + 2, km, kr)
        if do_fire_idx:
            fire_idx(j + NIDX - 2, (t - 2) % NIDX)

    for t in range(NIDX):
        fire_idx(t, t)
    plsc.subcore_barrier()
    for t in range(2):
        wait_idx(t, t)
        fire_gather(t, t, t)

    G = 12
    for t in range(G):
        step(t, t, t >= 2, True, t >= 2)

    def body(g, carry):
        for t in range(G):
            step(g * G + t, t, True, True, True)
        return carry

    lax.fori_loop(1, nch // G - 1, body, 0)
    for t in range(G):
        step(nch - G + t, t, t < G - 2, t < G - 2, t < G - 4)
    for t in range(G - NBUF, G):
        wait_scatter(nch - G + t, t % NIDX, t % NBUF)

    plsc.subcore_barrier()
    pltpu.sync_copy(acc.at[pl.ds(row0, ZR)],
                    out_hbm.at[cid, pl.ds(row0, ZR)])


def _sc_agg(s1, d1, hl):
    mesh = plsc.VectorSubcoreMesh(core_axis_name="c", subcore_axis_name="s")
    scratches = (
        [pltpu.VMEM((C,), jnp.int32) for _ in range(2 * NIDX)]
        + [pltpu.VMEM((C, H), jnp.float32) for _ in range(NBUF)]
        + [
            pltpu.VMEM_SHARED((NP, H), jnp.float32),
            pltpu.SemaphoreType.DMA((NIDX,)),
            pltpu.SemaphoreType.DMA((NBUF,)),
            pltpu.SemaphoreType.DMA((NBUF,)),
        ]
    )
    return pl.kernel(
        _sc_agg_body,
        out_type=jax.ShapeDtypeStruct((NC, NP, H), jnp.float32),
        mesh=mesh,
        scratch_types=scratches,
    )(s1, d1, hl)


def _tc_l1_body(degp_ref, x_ref, w_ref, hlp_ref, dis_ref):
    deg = degp_ref[0] + degp_ref[1] + 1.0
    dis = lax.rsqrt(deg)
    hl = jnp.dot(x_ref[...], w_ref[...], preferred_element_type=jnp.float32)
    hlp_ref[...] = hl * dis
    dis_ref[...] = dis


def _tc_l1(degp, x, W1):
    return pl.pallas_call(
        _tc_l1_body,
        grid=(N // RB,),
        in_specs=[
            pl.BlockSpec((NC, RB, 1), lambda i: (0, i, 0)),
            pl.BlockSpec((RB, D), lambda i: (i, 0)),
            pl.BlockSpec((D, H), lambda i: (0, 0)),
        ],
        out_specs=[
            pl.BlockSpec((RB, H), lambda i: (i, 0)),
            pl.BlockSpec((RB, 1), lambda i: (i, 0)),
        ],
        out_shape=[
            jax.ShapeDtypeStruct((N, H), jnp.float32),
            jax.ShapeDtypeStruct((N, 1), jnp.float32),
        ],
    )(degp, x, W1)


def _bn_relu(t, g, be):
    mu = jnp.mean(t, axis=0, keepdims=True)
    xc = t - mu
    var = jnp.mean(xc * xc, axis=0, keepdims=True)
    return jnp.maximum(xc * lax.rsqrt(var + 1e-5) * g + be, 0.0)


def _tc_mid_body(p_ref, hlp_ref, dis_ref, b_ref, g_ref, be_ref, w_ref,
                 out_ref):
    agg = p_ref[0, :N] + p_ref[1, :N] + hlp_ref[...]
    t = agg * dis_ref[...] + b_ref[...]
    h2 = _bn_relu(t, g_ref[...], be_ref[...])
    out_ref[...] = jnp.dot(h2, w_ref[...],
                           preferred_element_type=jnp.float32) * dis_ref[...]


def _tc_mid(p, hlp, dis, b, g, be, W):
    return pl.pallas_call(
        _tc_mid_body,
        out_shape=jax.ShapeDtypeStruct((N, H), jnp.float32),
    )(p, hlp, dis, b, g, be, W)


def _tc_final_body(p_ref, hlp_ref, dis_ref, b_ref, g_ref, be_ref, w_ref,
                   bf_ref, out_ref):
    agg = p_ref[0, :N] + p_ref[1, :N] + hlp_ref[...]
    t = agg * dis_ref[...] + b_ref[...]
    h3 = _bn_relu(t, g_ref[...], be_ref[...])
    out_ref[...] = jnp.dot(h3, w_ref[...],
                           preferred_element_type=jnp.float32) + bf_ref[...]


def _tc_final(p, hlp, dis, b, g, be, Wf, bf):
    return pl.pallas_call(
        _tc_final_body,
        out_shape=jax.ShapeDtypeStruct((N, H), jnp.float32),
    )(p, hlp, dis, b, g, be, Wf, bf)


def kernel(x, edge_index, W1, b1, g1, be1, W2, b2, g2, be2, Wf, bf):
    pad = EP - E
    s1 = jnp.concatenate(
        [edge_index[0], jnp.zeros((pad,), jnp.int32)])
    d1 = jnp.concatenate(
        [edge_index[1], jnp.full((pad,), N, jnp.int32)])
    b1r, g1r, be1r = b1.reshape(1, H), g1.reshape(1, H), be1.reshape(1, H)
    b2r, g2r, be2r = b2.reshape(1, H), g2.reshape(1, H), be2.reshape(1, H)
    bfr = bf.reshape(1, H)

    degp = _sc_hist(d1).reshape(NC, NP, 1)
    hlp1, dis = _tc_l1(degp, x, W1)
    p1 = _sc_agg(s1, d1, hlp1)
    hlp2 = _tc_mid(p1, hlp1, dis, b1r, g1r, be1r, W2)
    p2 = _sc_agg(s1, d1, hlp2)
    return _tc_final(p2, hlp2, dis, b2r, g2r, be2r, Wf, bfr)


# speedup vs baseline: 1.0739x; 1.0547x over previous
import jax
import jax.numpy as jnp
from jax import lax
from jax.experimental import pallas as pl
from jax.experimental.pallas import tpu as pltpu
from jax.experimental.pallas import tpu_sc as plsc

N = 10000
E = 320000
D = 128
H = 128

NC = 2
NS = 16
NW = NC * NS
C = 80
NCHUNK = 126
NCH0 = 228
NCH1 = 24
EP = NW * NCHUNK * C
NP = 10112
ZR = NP // NS
NBUF = 3
NIDX = 6
RB = 1000


NH = 6
ZR8 = 640


def _sc_hist_body(d1_hbm, out_hbm, *scr):
    dbufs = scr[0:NH]
    ones_v = scr[NH]
    stage = scr[NH + 1]
    acc = scr[NH + 2]
    sem_i = scr[NH + 3]

    cid = lax.axis_index("c")
    sid = lax.axis_index("s")
    wid = cid * NS + sid
    base = wid * (NCHUNK * C)
    for k in range(C // 16):
        ones_v[pl.ds(16 * k, 16)] = jnp.full((16,), 1.0, jnp.float32)
    for k in range(ZR8 // 16):
        stage[pl.ds(16 * k, 16)] = jnp.zeros((16,), jnp.float32)
    off0 = pl.multiple_of(sid * ZR, 8)
    pltpu.sync_copy(stage.at[pl.ds(0, ZR)], acc.at[pl.ds(off0, ZR)])

    def fire(j, m):
        src = d1_hbm.at[pl.ds(pl.multiple_of(base + j * C, 16), C)]
        pltpu.async_copy(src, dbufs[m], sem_i.at[m])

    def wait(j, m):
        src = d1_hbm.at[pl.ds(pl.multiple_of(base + j * C, 16), C)]
        pltpu.make_async_copy(src, dbufs[m], sem_i.at[m]).wait()

    for t in range(NH):
        fire(t, t)
    plsc.subcore_barrier()

    def step(j, t, last):
        wait(j, t)
        pltpu.sync_copy(ones_v, acc.at[dbufs[t]], add=True)
        if (not last) or (j + NH < NCHUNK):
            fire(j + NH, t)

    def body(g, carry):
        for t in range(NH):
            step(g * NH + t, t, False)
        return carry

    lax.fori_loop(0, NCHUNK // NH - 1, body, 0)
    for j in range(NCHUNK - NH, NCHUNK):
        step(j, j % NH, True)

    plsc.subcore_barrier()
    pltpu.sync_copy(acc.at[pl.ds(off0, ZR)], stage.at[pl.ds(0, ZR)])
    pltpu.sync_copy(stage.at[pl.ds(0, ZR)],
                    out_hbm.at[pl.ds(cid * NP + off0, ZR)])


def _sc_hist(d1):
    mesh = plsc.VectorSubcoreMesh(core_axis_name="c", subcore_axis_name="s")
    scratches = (
        [pltpu.VMEM((C,), jnp.int32) for _ in range(NH)]
        + [
            pltpu.VMEM((C,), jnp.float32),
            pltpu.VMEM((ZR8,), jnp.float32),
            pltpu.VMEM_SHARED((NP,), jnp.float32),
            pltpu.SemaphoreType.DMA((NH,)),
        ]
    )
    return pl.kernel(
        _sc_hist_body,
        out_type=jax.ShapeDtypeStruct((NC * NP,), jnp.float32),
        mesh=mesh,
        scratch_types=scratches,
    )(d1)


def _sc_agg_body(s1_hbm, d1_hbm, hl_hbm, out_hbm, *scr):
    sbufs = scr[0:NIDX]
    dbufs = scr[NIDX:2 * NIDX]
    rbufs = scr[2 * NIDX:2 * NIDX + NBUF]
    acc = scr[2 * NIDX + NBUF]
    sem_i = scr[2 * NIDX + NBUF + 1]
    sem_g = scr[2 * NIDX + NBUF + 2]
    sem_s = scr[2 * NIDX + NBUF + 3]

    cid = lax.axis_index("c")
    sid = lax.axis_index("s")
    nch = jnp.where(cid == 0, NCH0, NCH1)
    base = jnp.where(cid == 0, sid * (NCH0 * C),
                     NS * (NCH0 * C) + sid * (NCH1 * C))
    row0 = pl.multiple_of(sid * ZR, 8)

    def zfill(i, carry):
        for k in range(H // 16):
            rbufs[0][i, pl.ds(16 * k, 16)] = jnp.zeros((16,), jnp.float32)
        return carry

    lax.fori_loop(0, C, zfill, 0)
    for q in range(ZR // C):
        pltpu.sync_copy(rbufs[0], acc.at[pl.ds(row0 + C * q, C)])
    zr_tail = ZR - (ZR // C) * C
    if zr_tail:
        pltpu.sync_copy(rbufs[0].at[pl.ds(0, zr_tail)],
                        acc.at[pl.ds(row0 + (ZR // C) * C, zr_tail)])

    def _chunk(ref, j):
        return ref.at[pl.ds(pl.multiple_of(base + j * C, 16), C)]

    def fire_idx(j, m):
        pltpu.async_copy(_chunk(s1_hbm, j), sbufs[m], sem_i.at[m])
        pltpu.async_copy(_chunk(d1_hbm, j), dbufs[m], sem_i.at[m])

    def wait_idx(j, m):
        pltpu.make_async_copy(_chunk(s1_hbm, j), sbufs[m],
                              sem_i.at[m]).wait()
        pltpu.make_async_copy(_chunk(d1_hbm, j), dbufs[m],
                              sem_i.at[m]).wait()

    def fire_gather(j, m, r):
        pltpu.async_copy(hl_hbm.at[sbufs[m]], rbufs[r], sem_g.at[r])

    def wait_gather(j, m, r):
        pltpu.make_async_copy(hl_hbm.at[sbufs[m]], rbufs[r],
                              sem_g.at[r]).wait()

    def fire_scatter(j, m, r):
        pltpu.async_copy(rbufs[r], acc.at[dbufs[m]],
                         sem_s.at[r], add=True)

    def wait_scatter(j, m, r):
        pltpu.make_async_copy(rbufs[r], acc.at[dbufs[m]],
                              sem_s.at[r]).wait()

    def step(j, t, do_wait_sc, do_next_gather, do_fire_idx):
        m, r = t, t % NBUF
        wait_gather(j, m, r)
        fire_scatter(j, m, r)
        if do_next_gather:
            km, kr = (t + 2) % NIDX, (t + 2) % NBUF
            if do_wait_sc:
                wait_scatter(j - 1, (t - 1) % NIDX, kr)
            wait_idx(j + 2, km)
            fire_gather(j + 2, km, kr)
        if do_fire_idx:
            fire_idx(j + NIDX - 2, (t - 2) % NIDX)

    for t in range(NIDX):
        fire_idx(t, t)
    plsc.subcore_barrier()
    for t in range(2):
        wait_idx(t, t)
        fire_gather(t, t, t)

    G = NIDX
    for t in range(G):
        step(t, t, t >= 1, True, t >= 2)

    def body(g, carry):
        for t in range(G):
            step(g * G + t, t, True, True, True)
        return carry

    lax.fori_loop(1, nch // G - 1, body, 0)
    for t in range(G):
        step(nch - G + t, t, t < G - 2, t < G - 2, t < 2)
    for t in range(G - NBUF, G):
        wait_scatter(nch - G + t, t, t % NBUF)

    plsc.subcore_barrier()
    pltpu.sync_copy(acc.at[pl.ds(row0, ZR)],
                    out_hbm.at[cid, pl.ds(row0, ZR)])


def _sc_agg(s1, d1, hl):
    mesh = plsc.VectorSubcoreMesh(core_axis_name="c", subcore_axis_name="s")
    scratches = (
        [pltpu.VMEM((C,), jnp.int32) for _ in range(2 * NIDX)]
        + [pltpu.VMEM((C, H), jnp.float32) for _ in range(NBUF)]
        + [
            pltpu.VMEM_SHARED((NP, H), jnp.float32),
            pltpu.SemaphoreType.DMA((NIDX,)),
            pltpu.SemaphoreType.DMA((NBUF,)),
            pltpu.SemaphoreType.DMA((NBUF,)),
        ]
    )
    return pl.kernel(
        _sc_agg_body,
        out_type=jax.ShapeDtypeStruct((NC, NP, H), jnp.float32),
        mesh=mesh,
        scratch_types=scratches,
    )(s1, d1, hl)


def _tc_l1_body(degp_ref, x_ref, w_ref, hlp_ref, dis_ref):
    deg = degp_ref[0] + degp_ref[1] + 1.0
    dis = lax.rsqrt(deg)
    hl = jnp.dot(x_ref[...], w_ref[...], preferred_element_type=jnp.float32)
    hlp_ref[...] = hl * dis
    dis_ref[...] = dis


def _tc_l1(degp, x, W1):
    return pl.pallas_call(
        _tc_l1_body,
        grid=(N // RB,),
        in_specs=[
            pl.BlockSpec((NC, RB, 1), lambda i: (0, i, 0)),
            pl.BlockSpec((RB, D), lambda i: (i, 0)),
            pl.BlockSpec((D, H), lambda i: (0, 0)),
        ],
        out_specs=[
            pl.BlockSpec((RB, H), lambda i: (i, 0)),
            pl.BlockSpec((RB, 1), lambda i: (i, 0)),
        ],
        out_shape=[
            jax.ShapeDtypeStruct((N, H), jnp.float32),
            jax.ShapeDtypeStruct((N, 1), jnp.float32),
        ],
    )(degp, x, W1)


def _bn_relu(t, g, be):
    mu = jnp.mean(t, axis=0, keepdims=True)
    xc = t - mu
    var = jnp.mean(xc * xc, axis=0, keepdims=True)
    return jnp.maximum(xc * lax.rsqrt(var + 1e-5) * g + be, 0.0)


def _tc_mid_body(p_ref, hlp_ref, dis_ref, b_ref, g_ref, be_ref, w_ref,
                 out_ref):
    agg = p_ref[0, :N] + p_ref[1, :N] + hlp_ref[...]
    t = agg * dis_ref[...] + b_ref[...]
    h2 = _bn_relu(t, g_ref[...], be_ref[...])
    out_ref[...] = jnp.dot(h2, w_ref[...],
                           preferred_element_type=jnp.float32) * dis_ref[...]


def _tc_mid(p, hlp, dis, b, g, be, W):
    return pl.pallas_call(
        _tc_mid_body,
        out_shape=jax.ShapeDtypeStruct((N, H), jnp.float32),
    )(p, hlp, dis, b, g, be, W)


def _tc_final_body(p_ref, hlp_ref, dis_ref, b_ref, g_ref, be_ref, w_ref,
                   bf_ref, out_ref):
    agg = p_ref[0, :N] + p_ref[1, :N] + hlp_ref[...]
    t = agg * dis_ref[...] + b_ref[...]
    h3 = _bn_relu(t, g_ref[...], be_ref[...])
    out_ref[...] = jnp.dot(h3, w_ref[...],
                           preferred_element_type=jnp.float32) + bf_ref[...]


def _tc_final(p, hlp, dis, b, g, be, Wf, bf):
    return pl.pallas_call(
        _tc_final_body,
        out_shape=jax.ShapeDtypeStruct((N, H), jnp.float32),
    )(p, hlp, dis, b, g, be, Wf, bf)


def kernel(x, edge_index, W1, b1, g1, be1, W2, b2, g2, be2, Wf, bf):
    pad = EP - E
    s1 = jnp.concatenate(
        [edge_index[0], jnp.zeros((pad,), jnp.int32)])
    d1 = jnp.concatenate(
        [edge_index[1], jnp.full((pad,), N, jnp.int32)])
    b1r, g1r, be1r = b1.reshape(1, H), g1.reshape(1, H), be1.reshape(1, H)
    b2r, g2r, be2r = b2.reshape(1, H), g2.reshape(1, H), be2.reshape(1, H)
    bfr = bf.reshape(1, H)

    degp = _sc_hist(d1).reshape(NC, NP, 1)
    hlp1, dis = _tc_l1(degp, x, W1)
    p1 = _sc_agg(s1, d1, hlp1)
    hlp2 = _tc_mid(p1, hlp1, dis, b1r, g1r, be1r, W2)
    p2 = _sc_agg(s1, d1, hlp2)
    return _tc_final(p2, hlp2, dis, b2r, g2r, be2r, Wf, bfr)
